# baseline (speedup 1.0000x reference)
import functools

import jax
import jax.numpy as jnp
from jax import lax
from jax.experimental import pallas as pl
from jax.experimental.pallas import tpu as pltpu
from jax.experimental.pallas import tpu_sc as plsc

NC = 2
NS = 16
NW = NC * NS
L = 16
PW = 128
NPAN = 7813
PPW = 245
NBUF = 4
NPPAD = 248
RB = 4


def _make_sc_gather(V, D, B):
    mesh = plsc.VectorSubcoreMesh(core_axis_name="c", subcore_axis_name="s")

    @functools.partial(
        pl.kernel,
        mesh=mesh,
        compiler_params=pltpu.CompilerParams(needs_layout_passes=False),
        out_type=jax.ShapeDtypeStruct((B + L, PW), jnp.float32),
        scratch_types=[
            pltpu.VMEM((B // PW, PW), jnp.int32),
            pltpu.VMEM((B + L,), jnp.int32),
            pltpu.VMEM((B + L,), jnp.int32),
            pltpu.VMEM((NBUF, D, PW), jnp.float32),
            pltpu.VMEM((RB, L, PW), jnp.float32),
            [pltpu.SemaphoreType.DMA] * NBUF,
            pltpu.SemaphoreType.DMA,
        ],
    )
    def gather_k(y_hbm, embT_hbm, g_hbm, idx_v, mi_v, mr_v, panels_v,
                 rowsbuf_v, psems, ssem):
        wid = lax.axis_index("s") * NC + lax.axis_index("c")
        p_lo = wid * PPW
        lanes = lax.iota(jnp.int32, L)

        pltpu.sync_copy(y_hbm, idx_v)

        def scan_body(g, cnt):
            iv = idx_v[lax.shift_right_logical(g, 3),
                       pl.ds(pl.multiple_of((g & 7) * L, L), L)]
            b = lax.shift_right_logical(iv, 7)
            m = (b >= p_lo) & (b < p_lo + PPW)
            plsc.store_compressed(mi_v.at[pl.ds(cnt, L)], iv, mask=m)
            rv = g * L + lanes
            plsc.store_compressed(mr_v.at[pl.ds(cnt, L)], rv, mask=m)
            return cnt + plsc.all_reduce_population_count(m)[0]

        cnt = lax.fori_loop(0, B // L, scan_body, jnp.int32(0))
        nv = lax.shift_right_logical(cnt + (L - 1), 4)

        def fire(p, buf, sem):
            pe = jnp.minimum(p_lo + p, NPAN - 1)
            start = pl.multiple_of(pe * PW, PW)
            pltpu.async_copy(
                embT_hbm.at[:, pl.ds(start, PW)], panels_v.at[buf], sem
            )

        def drain_panel(buf, sem):
            pltpu.make_async_copy(
                embT_hbm.at[:, pl.ds(0, PW)], panels_v.at[buf], sem
            ).wait()

        def drain_scatter():
            pltpu.make_async_copy(
                rowsbuf_v.at[0], g_hbm.at[pl.ds(0, L)], ssem
            ).wait()

        def extract_panel(p, buf, sid0):
            pg = p_lo + p

            def vreg_body(jv, sid):
                off = pl.multiple_of(jv * L, L)
                iv16 = mi_v[pl.ds(off, L)]
                rv16 = mr_v[pl.ds(off, L)]
                valid = (off + lanes) < cnt
                m = (lax.shift_right_logical(iv16, 7) == pg) & valid
                hc = plsc.all_reduce_population_count(m)[0]
                par = sid & (RB - 1)

                @pl.when(hc > 0)
                def _():
                    @pl.when(sid >= RB)
                    def _():
                        drain_scatter()

                    for j in range(L):
                        i_j = iv16[j]
                        hit_j = (lax.shift_right_logical(i_j, 7) == pg) & (
                            (off + j) < cnt)

                        @pl.when(hit_j)
                        def _():
                            col = jnp.full((L,), i_j & (PW - 1), jnp.int32)
                            for kk in range(D // L):
                                q = plsc.load_gather(
                                    panels_v.at[buf],
                                    [lanes + (L * kk), col],
                                )
                                rowsbuf_v[par, j, pl.ds(L * kk, L)] = q

                    idxreg = jnp.where(m, rv16, B + lanes)
                    pltpu.async_copy(
                        rowsbuf_v.at[par], g_hbm.at[idxreg], ssem
                    )

                return sid + jnp.where(hc > 0, 1, 0).astype(jnp.int32)

            return lax.fori_loop(0, nv, vreg_body, sid0)

        for pp in range(NBUF - 1):
            fire(jnp.int32(pp), pp, psems[pp])

        def quad_body(q, sid):
            p0 = q * NBUF
            for pp in range(NBUF):
                p = p0 + pp
                nb = (pp + NBUF - 1) % NBUF

                @pl.when(p + NBUF - 1 < NPPAD)
                def _():
                    fire(p + NBUF - 1, nb, psems[nb])

                drain_panel(pp, psems[pp])
                sid = extract_panel(p, pp, sid)
            return sid

        sid = lax.fori_loop(0, NPPAD // NBUF, quad_body, jnp.int32(0))

        def final_drain(i, carry):
            drain_scatter()
            return carry

        lax.fori_loop(0, jnp.minimum(sid, RB), final_drain, 0)

    return gather_k


def _tc_combine(zT, g):
    D, B = zT.shape
    blk = 2048

    def body(z_ref, g_ref, o_ref):
        zb = z_ref[...]
        s = jnp.sum(zb * zb, axis=0, keepdims=True)
        eye = jnp.eye(D, dtype=jnp.float32)
        gt = lax.dot_general(
            eye, g_ref[..., :D], (((1,), (1,)), ((), ())),
            precision=lax.Precision.HIGHEST,
        )
        o_ref[...] = zb * lax.rsqrt(s) + gt

    return pl.pallas_call(
        body,
        grid=(B // blk,),
        in_specs=[
            pl.BlockSpec((D, blk), lambda i: (0, i)),
            pl.BlockSpec((blk, PW), lambda i: (i, 0)),
        ],
        out_specs=pl.BlockSpec((D, blk), lambda i: (0, i)),
        out_shape=jax.ShapeDtypeStruct((D, B), jnp.float32),
    )(zT, g)


def kernel(z, y, embedding):
    B, D = z.shape
    V = embedding.shape[0]
    y2 = y.astype(jnp.int32).reshape(B // PW, PW)
    g = _make_sc_gather(V, D, B)(y2, embedding.T)
    outT = _tc_combine(z.T, g)
    return outT.T


# speedup vs baseline: 1.1072x; 1.1072x over previous
import functools

import jax
import jax.numpy as jnp
from jax import lax
from jax.experimental import pallas as pl
from jax.experimental.pallas import tpu as pltpu
from jax.experimental.pallas import tpu_sc as plsc

NC = 2
NS = 16
NW = NC * NS
L = 16
PW = 128
NPAN = 7813
PPW = 245
GP = 4
NGRP = 62
RB = 4


def _make_sc_gather(V, D, B):
    mesh = plsc.VectorSubcoreMesh(core_axis_name="c", subcore_axis_name="s")

    @functools.partial(
        pl.kernel,
        mesh=mesh,
        compiler_params=pltpu.CompilerParams(needs_layout_passes=False),
        out_type=jax.ShapeDtypeStruct((B + L, PW), jnp.float32),
        scratch_types=[
            pltpu.VMEM((B // PW, PW), jnp.int32),
            pltpu.VMEM((B + L,), jnp.int32),
            pltpu.VMEM((B + L,), jnp.int32),
            pltpu.VMEM((2 * GP, D, PW), jnp.float32),
            pltpu.VMEM((RB, L, PW), jnp.float32),
            [pltpu.SemaphoreType.DMA] * (2 * GP),
            pltpu.SemaphoreType.DMA,
        ],
    )
    def gather_k(y_hbm, embT_hbm, g_hbm, idx_v, mi_v, mr_v, panels_v,
                 rowsbuf_v, psems, ssem):
        wid = lax.axis_index("s") * NC + lax.axis_index("c")
        p_lo = wid * PPW
        lanes = lax.iota(jnp.int32, L)

        pltpu.sync_copy(y_hbm, idx_v)

        def scan_body(g, cnt):
            iv = idx_v[lax.shift_right_logical(g, 3),
                       pl.ds(pl.multiple_of((g & 7) * L, L), L)]
            b = lax.shift_right_logical(iv, 7)
            m = (b >= p_lo) & (b < p_lo + PPW)
            plsc.store_compressed(mi_v.at[pl.ds(cnt, L)], iv, mask=m)
            rv = g * L + lanes
            plsc.store_compressed(mr_v.at[pl.ds(cnt, L)], rv, mask=m)
            return cnt + plsc.all_reduce_population_count(m)[0]

        cnt = lax.fori_loop(0, B // L, scan_body, jnp.int32(0))
        nv = lax.shift_right_logical(cnt + (L - 1), 4)

        def fire(p, buf, sem):
            pe = jnp.minimum(p_lo + p, NPAN - 1)
            start = pl.multiple_of(pe * PW, PW)
            pltpu.async_copy(
                embT_hbm.at[:, pl.ds(start, PW)], panels_v.at[buf], sem
            )

        def drain_panel(buf, sem):
            pltpu.make_async_copy(
                embT_hbm.at[:, pl.ds(0, PW)], panels_v.at[buf], sem
            ).wait()

        def drain_scatter():
            pltpu.make_async_copy(
                rowsbuf_v.at[0], g_hbm.at[pl.ds(0, L)], ssem
            ).wait()

        def extract_group(grp, gbase, sid0):
            pg0 = p_lo + grp * GP

            def handle(off, iv16, rv16, m, hc, sid):
                par = sid & (RB - 1)

                @pl.when(hc > 0)
                def _():
                    @pl.when(sid >= RB)
                    def _():
                        drain_scatter()

                    for j in range(L):
                        i_j = iv16[j]
                        b_j = lax.shift_right_logical(i_j, 7)
                        hit_j = (b_j >= pg0) & (b_j < pg0 + GP) & (
                            (off + j) < cnt)

                        @pl.when(hit_j)
                        def _():
                            pb = jnp.full((L,), gbase + (b_j - pg0),
                                          jnp.int32)
                            col = jnp.full((L,), i_j & (PW - 1), jnp.int32)
                            for kk in range(D // L):
                                q = plsc.load_gather(
                                    panels_v,
                                    [pb, lanes + (L * kk), col],
                                )
                                rowsbuf_v[par, j, pl.ds(L * kk, L)] = q

                    idxreg = jnp.where(m, rv16, B + lanes)
                    pltpu.async_copy(
                        rowsbuf_v.at[par], g_hbm.at[idxreg], ssem
                    )

                return sid + jnp.where(hc > 0, 1, 0).astype(jnp.int32)

            def vreg_body(jv2, sid):
                for u in range(2):
                    off = pl.multiple_of((jv2 * 2 + u) * L, L)
                    iv16 = mi_v[pl.ds(off, L)]
                    rv16 = mr_v[pl.ds(off, L)]
                    b = lax.shift_right_logical(iv16, 7)
                    valid = (off + lanes) < cnt
                    m = (b >= pg0) & (b < pg0 + GP) & valid
                    hc = plsc.all_reduce_population_count(m)[0]
                    sid = handle(off, iv16, rv16, m, hc, sid)
                return sid

            nv2 = lax.shift_right_logical(cnt + (2 * L - 1), 5)
            return lax.fori_loop(0, nv2, vreg_body, sid0)

        for pp in range(2 * GP):
            fire(jnp.int32(pp), pp, psems[pp])

        def group_body(grp, sid):
            gbase_l = [0, GP]
            for gpar in range(2):
                g2 = grp * 2 + gpar
                gbase = gbase_l[gpar]
                for pp in range(GP):
                    drain_panel(gbase + pp, psems[gbase + pp])
                sid = extract_group(g2, gbase, sid)

                @pl.when(g2 + 2 < NGRP)
                def _():
                    for pp in range(GP):
                        fire((g2 + 2) * GP + pp, gbase + pp,
                             psems[gbase + pp])

            return sid

        sid = lax.fori_loop(0, NGRP // 2, group_body, jnp.int32(0))

        def final_drain(i, carry):
            drain_scatter()
            return carry

        lax.fori_loop(0, jnp.minimum(sid, RB), final_drain, 0)

    return gather_k


def _tc_combine(zT, g):
    D, B = zT.shape
    blk = 2048

    def body(z_ref, g_ref, o_ref):
        zb = z_ref[...]
        s = jnp.sum(zb * zb, axis=0, keepdims=True)
        eye = jnp.eye(D, dtype=jnp.float32)
        gt = lax.dot_general(
            eye, g_ref[..., :D], (((1,), (1,)), ((), ())),
            precision=lax.Precision.HIGHEST,
        )
        o_ref[...] = zb * lax.rsqrt(s) + gt

    return pl.pallas_call(
        body,
        grid=(B // blk,),
        in_specs=[
            pl.BlockSpec((D, blk), lambda i: (0, i)),
            pl.BlockSpec((blk, PW), lambda i: (i, 0)),
        ],
        out_specs=pl.BlockSpec((D, blk), lambda i: (0, i)),
        out_shape=jax.ShapeDtypeStruct((D, B), jnp.float32),
    )(zT, g)


def kernel(z, y, embedding):
    B, D = z.shape
    V = embedding.shape[0]
    y2 = y.astype(jnp.int32).reshape(B // PW, PW)
    g = _make_sc_gather(V, D, B)(y2, embedding.T)
    outT = _tc_combine(z.T, g)
    return outT.T


# speedup vs baseline: 4.4380x; 4.0085x over previous
import functools

import jax
import jax.numpy as jnp
from jax import lax
from jax.experimental import pallas as pl
from jax.experimental.pallas import tpu as pltpu
from jax.experimental.pallas import tpu_sc as plsc

NC = 2
NS = 16
NW = NC * NS
L = 16
PW = 128
NBUF = 8
HALF = 256


def _make_sc_gather(V, D, B):
    bpw = B // NW

    mesh = plsc.VectorSubcoreMesh(core_axis_name="c", subcore_axis_name="s")

    @functools.partial(
        pl.kernel,
        mesh=mesh,
        compiler_params=pltpu.CompilerParams(needs_layout_passes=False),
        out_type=jax.ShapeDtypeStruct((B, PW), jnp.float32),
        scratch_types=[
            pltpu.VMEM((bpw // PW, PW), jnp.int32),
            pltpu.VMEM((NBUF, D, PW), jnp.float32),
            pltpu.VMEM((HALF, PW), jnp.float32),
            pltpu.VMEM((D, HALF), jnp.float32),
            [pltpu.SemaphoreType.DMA] * NBUF,
        ],
    )
    def gather_k(y_hbm, embT_hbm, zT_hbm, g_hbm, idx_v, panels_v, rows_v,
                 z_v, sems):
        wid = lax.axis_index("s") * NC + lax.axis_index("c")
        base = wid * bpw
        pltpu.sync_copy(y_hbm.at[wid], idx_v)

        lanes = lax.iota(jnp.int32, L)
        perms = [lax.bitwise_xor(lanes, jnp.int32(k)) for k in (8, 4, 2, 1)]

        def scalar_idx(r):
            g = lax.shift_right_logical(r, 4)
            vec = idx_v[lax.shift_right_logical(g, 3),
                        pl.ds(pl.multiple_of((g & 7) * L, L), L)]
            j = jnp.full((L,), r & (L - 1), jnp.int32)
            return vec.at[j].get(mode="promise_in_bounds")[0]

        def fire(r, buf, sem):
            i = scalar_idx(r)
            start = pl.multiple_of(i & ~jnp.int32(PW - 1), PW)
            pltpu.async_copy(
                embT_hbm.at[:, pl.ds(start, PW)], panels_v.at[buf], sem
            )

        def drain(buf, sem):
            pltpu.make_async_copy(
                embT_hbm.at[:, pl.ds(0, PW)], panels_v.at[buf], sem
            ).wait()

        def extract(r, buf):
            i = scalar_idx(r)
            col = jnp.full((L,), i & (PW - 1), jnp.int32)
            rl = jnp.full((L,), r & (HALF - 1), jnp.int32)
            zq = []
            for k in range(D // L):
                zq.append(plsc.load_gather(z_v, [lanes + (L * k), rl]))
            s_vec = zq[0] * zq[0]
            for k in range(1, D // L):
                s_vec = s_vec + zq[k] * zq[k]
            for perm in perms:
                s_vec = s_vec + s_vec.at[perm].get(mode="promise_in_bounds")
            iv = lax.bitcast_convert_type(s_vec, jnp.int32)
            iv = jnp.int32(0x5F3759DF) - lax.shift_right_logical(iv, 1)
            yv = lax.bitcast_convert_type(iv, jnp.float32)
            half_s = s_vec * 0.5
            for _ in range(3):
                yv = yv * (1.5 - half_s * yv * yv)
            for k in range(D // L):
                row_idx = lanes + (L * k)
                q = plsc.load_gather(panels_v.at[buf], [row_idx, col])
                rows_v[r & (HALF - 1), pl.ds(L * k, L)] = q + zq[k] * yv

        for h in range(bpw // HALF):
            r_lo = h * HALF
            r_hi = r_lo + HALF
            pltpu.sync_copy(
                zT_hbm.at[:, pl.ds(pl.multiple_of(base + r_lo, HALF), HALF)],
                z_v,
            )
            for p in range(NBUF - 1):
                fire(jnp.int32(r_lo + p), p, sems[p])

            def quad_body(rq, carry):
                r0 = r_lo + rq * NBUF
                for p in range(NBUF):
                    r = r0 + p
                    nb = (p + NBUF - 1) % NBUF

                    @pl.when(r + NBUF - 1 < r_hi)
                    def _():
                        fire(r + NBUF - 1, nb, sems[nb])

                    drain(p, sems[p])
                    extract(r, p)
                return carry

            lax.fori_loop(0, HALF // NBUF, quad_body, 0)
            pltpu.sync_copy(rows_v, g_hbm.at[pl.ds(base + r_lo, HALF)])

    return gather_k


def kernel(z, y, embedding):
    B, D = z.shape
    V = embedding.shape[0]
    bpw = B // NW
    y3 = y.astype(jnp.int32).reshape(NW, bpw // PW, PW)
    g = _make_sc_gather(V, D, B)(y3, embedding.T, z.T)
    return g[:, :D]


# speedup vs baseline: 4.5365x; 1.0222x over previous
import functools

import jax
import jax.numpy as jnp
from jax import lax
from jax.experimental import pallas as pl
from jax.experimental.pallas import tpu as pltpu
from jax.experimental.pallas import tpu_sc as plsc

NC = 2
NS = 16
NW = NC * NS
L = 16
PW = 128
NBUF = 8
HALF = 256


def _make_sc_gather(V, D, B):
    bpw = B // NW

    mesh = plsc.VectorSubcoreMesh(core_axis_name="c", subcore_axis_name="s")

    @functools.partial(
        pl.kernel,
        mesh=mesh,
        compiler_params=pltpu.CompilerParams(needs_layout_passes=False),
        out_type=jax.ShapeDtypeStruct((D, B), jnp.float32),
        scratch_types=[
            pltpu.VMEM((bpw // PW, PW), jnp.int32),
            pltpu.VMEM((NBUF, D, PW), jnp.float32),
            pltpu.VMEM((D, HALF), jnp.float32),
            pltpu.VMEM((D, HALF), jnp.float32),
            [pltpu.SemaphoreType.DMA] * NBUF,
        ],
    )
    def gather_k(y_hbm, embT_hbm, zT_hbm, g_hbm, idx_v, panels_v, rows_v,
                 z_v, sems):
        wid = lax.axis_index("s") * NC + lax.axis_index("c")
        base = wid * bpw
        pltpu.sync_copy(y_hbm.at[wid], idx_v)

        lanes = lax.iota(jnp.int32, L)
        perms = [lax.bitwise_xor(lanes, jnp.int32(k)) for k in (8, 4, 2, 1)]

        def scalar_idx(r):
            g = lax.shift_right_logical(r, 4)
            vec = idx_v[lax.shift_right_logical(g, 3),
                        pl.ds(pl.multiple_of((g & 7) * L, L), L)]
            j = jnp.full((L,), r & (L - 1), jnp.int32)
            return vec.at[j].get(mode="promise_in_bounds")[0]

        def fire(r, buf, sem):
            i = scalar_idx(r)
            start = pl.multiple_of(i & ~jnp.int32(PW - 1), PW)
            pltpu.async_copy(
                embT_hbm.at[:, pl.ds(start, PW)], panels_v.at[buf], sem
            )

        def drain(buf, sem):
            pltpu.make_async_copy(
                embT_hbm.at[:, pl.ds(0, PW)], panels_v.at[buf], sem
            ).wait()

        def extract(r, buf):
            i = scalar_idx(r)
            col = jnp.full((L,), i & (PW - 1), jnp.int32)
            rl = jnp.full((L,), r & (HALF - 1), jnp.int32)
            zq = []
            for k in range(D // L):
                zq.append(plsc.load_gather(z_v, [lanes + (L * k), rl]))
            s_vec = zq[0] * zq[0]
            for k in range(1, D // L):
                s_vec = s_vec + zq[k] * zq[k]
            for perm in perms:
                s_vec = s_vec + s_vec.at[perm].get(mode="promise_in_bounds")
            iv = lax.bitcast_convert_type(s_vec, jnp.int32)
            iv = jnp.int32(0x5F3759DF) - lax.shift_right_logical(iv, 1)
            yv = lax.bitcast_convert_type(iv, jnp.float32)
            half_s = s_vec * 0.5
            for _ in range(3):
                yv = yv * (1.5 - half_s * yv * yv)
            for k in range(D // L):
                row_idx = lanes + (L * k)
                q = plsc.load_gather(panels_v.at[buf], [row_idx, col])
                plsc.store_scatter(rows_v, [row_idx, rl], q + zq[k] * yv)

        for h in range(bpw // HALF):
            r_lo = h * HALF
            r_hi = r_lo + HALF
            pltpu.sync_copy(
                zT_hbm.at[:, pl.ds(pl.multiple_of(base + r_lo, HALF), HALF)],
                z_v,
            )
            for p in range(NBUF - 1):
                fire(jnp.int32(r_lo + p), p, sems[p])

            def quad_body(rq, carry):
                r0 = r_lo + rq * NBUF
                for p in range(NBUF):
                    r = r0 + p
                    nb = (p + NBUF - 1) % NBUF

                    @pl.when(r + NBUF - 1 < r_hi)
                    def _():
                        fire(r + NBUF - 1, nb, sems[nb])

                    drain(p, sems[p])
                    extract(r, p)
                return carry

            lax.fori_loop(0, HALF // NBUF, quad_body, 0)
            pltpu.sync_copy(
                rows_v,
                g_hbm.at[:, pl.ds(pl.multiple_of(base + r_lo, HALF), HALF)],
            )

    return gather_k


def kernel(z, y, embedding):
    B, D = z.shape
    V = embedding.shape[0]
    bpw = B // NW
    y3 = y.astype(jnp.int32).reshape(NW, bpw // PW, PW)
    outT = _make_sc_gather(V, D, B)(y3, embedding.T, z.T)
    return outT.T
